# baseline (speedup 1.0000x reference)
import jax
import jax.numpy as jnp
from jax import lax
from jax.experimental import pallas as pl
from jax.experimental.pallas import tpu as pltpu, tpu_sc as plsc

NUM_FIELDS = 26
VOCAB = 100000
DIM = 64
BATCH = 4096

NC, NS, L = 2, 16, 16
FPC = NUM_FIELDS // NC
DPS = DIM // NS
NPIECE = 3
PS = (33408, 33408, 33184)
PO = (0, 33408, 66816)
NPOS = FPC * DPS * NPIECE
CHUNKS = BATCH // L
UNROLL = 8


def _acc_body(idx_hbm, wt_hbm, part_hbm, idxv, b0, b1, b2, acc, s0, s1, s2):
    cid = lax.axis_index("c")
    sid = lax.axis_index("s")
    bufs = (b0, b1, b2)
    sems = (s0, s1, s2)

    def zstep(i, _):
        acc[pl.ds(i * L, L)] = jnp.zeros((L,), jnp.float32)
        return 0

    lax.fori_loop(0, DPS * BATCH // L, zstep, 0)

    iota = lax.iota(jnp.int32, L)

    def src(pos, k):
        fi = pos // (DPS * NPIECE)
        dslot = (pos // NPIECE) % DPS
        f = cid * FPC + fi
        d = sid * DPS + dslot
        return wt_hbm.at[f, d, pl.ds(PO[k], PS[k])]

    pltpu.async_copy(src(0, 0), bufs[0], sems[0])
    pltpu.async_copy(src(1, 1), bufs[1], sems[1])

    def compute(dslot, k):
        lo = jnp.int32(PO[k])
        hi = jnp.int32(PO[k] + PS[k])
        base_f = dslot * BATCH
        buf = bufs[k]

        def kstep(kk, _):
            for j in range(UNROLL):
                b0_ = kk * (L * UNROLL) + j * L
                v = idxv[pl.ds(b0_, L)]
                fidx = iota + (base_f + b0_)
                if k == 0:
                    m = v < hi
                elif k == NPIECE - 1:
                    m = v >= lo
                else:
                    m = jnp.logical_and(v >= lo, v < hi)
                col = jnp.where(m, v - lo, 0)
                val = plsc.load_gather(buf, [col], mask=m)
                plsc.addupdate_scatter(acc, [fidx], val, mask=m)
            return 0

        lax.fori_loop(0, CHUNKS // UNROLL, kstep, 0)

    def pos_step(pos, _):
        fi = pos // (DPS * NPIECE)
        piece = pos % NPIECE
        f = cid * FPC + fi

        @pl.when(pos % (DPS * NPIECE) == 0)
        def _():
            pltpu.sync_copy(idx_hbm.at[f], idxv)

        for k in range(NPIECE):
            @pl.when(piece == k)
            def _(k=k):
                nk = (k + 2) % NPIECE
                dslot = (pos // NPIECE) % DPS

                @pl.when(pos + 2 < NPOS)
                def _():
                    pltpu.async_copy(src(pos + 2, nk), bufs[nk], sems[nk])

                pltpu.make_async_copy(src(pos, k), bufs[k], sems[k]).wait()
                compute(dslot, k)

        return 0

    lax.fori_loop(0, NPOS, pos_step, 0)

    for dslot in range(DPS):
        d = sid * DPS + dslot
        pltpu.sync_copy(
            acc.at[pl.ds(dslot * BATCH, BATCH)], part_hbm.at[cid, d]
        )


def _combine_body(part_hbm, out_hbm, p0v, p1v, ov):
    wid = lax.axis_index("s") * NC + lax.axis_index("c")
    scale = jnp.float32(1.0 / NUM_FIELDS)
    pltpu.sync_copy(part_hbm.at[0, :, pl.ds(wid * 128, 128)], p0v)
    pltpu.sync_copy(part_hbm.at[1, :, pl.ds(wid * 128, 128)], p1v)

    def rstep(r, _):
        for j in range(128 // L):
            s = pl.ds(j * L, L)
            ov[r, s] = (p0v[r, s] + p1v[r, s]) * scale
        return 0

    lax.fori_loop(0, DIM, rstep, 0)
    pltpu.sync_copy(ov, out_hbm.at[:, pl.ds(wid * 128, 128)])


@jax.jit
def _multi_embedding(idx2d, wt):
    mesh = plsc.VectorSubcoreMesh(core_axis_name="c", subcore_axis_name="s")
    k1 = pl.kernel(
        _acc_body,
        out_type=jax.ShapeDtypeStruct((NC, DIM, BATCH), jnp.float32),
        mesh=mesh,
        scratch_types=[
            pltpu.VMEM((BATCH,), jnp.int32),
            pltpu.VMEM((PS[0],), jnp.float32),
            pltpu.VMEM((PS[1],), jnp.float32),
            pltpu.VMEM((PS[2],), jnp.float32),
            pltpu.VMEM((DPS * BATCH,), jnp.float32),
            pltpu.SemaphoreType.DMA,
            pltpu.SemaphoreType.DMA,
            pltpu.SemaphoreType.DMA,
        ],
        compiler_params=pltpu.CompilerParams(
            use_tc_tiling_on_sc=True, needs_layout_passes=False
        ),
    )
    k2 = pl.kernel(
        _combine_body,
        out_type=jax.ShapeDtypeStruct((DIM, BATCH), jnp.float32),
        mesh=mesh,
        scratch_types=[
            pltpu.VMEM((DIM, 128), jnp.float32),
            pltpu.VMEM((DIM, 128), jnp.float32),
            pltpu.VMEM((DIM, 128), jnp.float32),
        ],
        compiler_params=pltpu.CompilerParams(
            use_tc_tiling_on_sc=True, needs_layout_passes=False
        ),
    )
    part = k1(idx2d, wt)
    return k2(part)


def kernel(xs, W):
    idx2d = xs[:, :, 0].astype(jnp.int32)
    wt = jnp.transpose(W, (0, 2, 1))
    out_t = _multi_embedding(idx2d, wt)
    return jnp.transpose(out_t)


# speedup vs baseline: 1.3481x; 1.3481x over previous
import jax
import jax.numpy as jnp
from jax import lax
from jax.experimental import pallas as pl
from jax.experimental.pallas import tpu as pltpu, tpu_sc as plsc

NUM_FIELDS = 26
VOCAB = 100000
DIM = 64
BATCH = 4096

NC, NS, L = 2, 16, 16
FPC = NUM_FIELDS // NC
DPS = DIM // NS
NPIECE = 2
PS = (50048, 49952)
PO = (0, 50048)
NPOS = FPC * DPS * NPIECE
CHUNKS = BATCH // L
UNROLL = 8


def _acc_body(idx_hbm, wt_hbm, part_hbm, idxv, b0, b1, acc, s0, s1):
    cid = lax.axis_index("c")
    sid = lax.axis_index("s")
    bufs = (b0, b1)
    sems = (s0, s1)

    def zstep(i, _):
        acc[pl.ds(i * L, L)] = jnp.zeros((L,), jnp.float32)
        return 0

    lax.fori_loop(0, DPS * BATCH // L, zstep, 0)

    def src(pos, k):
        fi = pos // (DPS * NPIECE)
        dslot = (pos // NPIECE) % DPS
        f = cid * FPC + fi
        d = sid * DPS + dslot
        return wt_hbm.at[f, d, pl.ds(PO[k], PS[k])]

    pltpu.async_copy(src(0, 0), bufs[0], sems[0])
    pltpu.async_copy(src(1, 1), bufs[1], sems[1])

    def compute(dslot, k):
        lo = jnp.int32(PO[k])
        hi = jnp.int32(PO[k] + PS[k])
        base_f = dslot * BATCH
        buf = bufs[k]
        zero = jnp.zeros((L,), jnp.float32)

        def kstep(kk, _):
            for j in range(UNROLL):
                b0_ = kk * (L * UNROLL) + j * L
                v = idxv[pl.ds(b0_, L)]
                if k == 0:
                    m = v < hi
                else:
                    m = v >= lo
                col = jnp.where(m, v - lo, 0)
                val = plsc.load_gather(buf, [col], mask=m)
                plsc.addupdate(acc.at[pl.ds(base_f + b0_, L)],
                               jnp.where(m, val, zero))
            return 0

        lax.fori_loop(0, CHUNKS // UNROLL, kstep, 0)

    def pos_step(pos, _):
        fi = pos // (DPS * NPIECE)
        piece = pos % NPIECE
        f = cid * FPC + fi

        @pl.when(pos % (DPS * NPIECE) == 0)
        def _():
            pltpu.sync_copy(idx_hbm.at[f], idxv)

        for k in range(NPIECE):
            @pl.when(piece == k)
            def _(k=k):
                dslot = (pos // NPIECE) % DPS
                pltpu.make_async_copy(src(pos, k), bufs[k], sems[k]).wait()
                compute(dslot, k)

                @pl.when(pos + 2 < NPOS)
                def _():
                    pltpu.async_copy(src(pos + 2, k), bufs[k], sems[k])

        return 0

    lax.fori_loop(0, NPOS, pos_step, 0)

    for dslot in range(DPS):
        d = sid * DPS + dslot
        pltpu.sync_copy(
            acc.at[pl.ds(dslot * BATCH, BATCH)], part_hbm.at[cid, d]
        )


def _combine_body(part_hbm, out_hbm, p0v, p1v, ov):
    wid = lax.axis_index("s") * NC + lax.axis_index("c")
    scale = jnp.float32(1.0 / NUM_FIELDS)
    pltpu.sync_copy(part_hbm.at[0, :, pl.ds(wid * 128, 128)], p0v)
    pltpu.sync_copy(part_hbm.at[1, :, pl.ds(wid * 128, 128)], p1v)

    def rstep(r, _):
        for j in range(128 // L):
            s = pl.ds(j * L, L)
            ov[r, s] = (p0v[r, s] + p1v[r, s]) * scale
        return 0

    lax.fori_loop(0, DIM, rstep, 0)
    pltpu.sync_copy(ov, out_hbm.at[:, pl.ds(wid * 128, 128)])


@jax.jit
def _multi_embedding(idx2d, wt):
    mesh = plsc.VectorSubcoreMesh(core_axis_name="c", subcore_axis_name="s")
    k1 = pl.kernel(
        _acc_body,
        out_type=jax.ShapeDtypeStruct((NC, DIM, BATCH), jnp.float32),
        mesh=mesh,
        scratch_types=[
            pltpu.VMEM((BATCH,), jnp.int32),
            pltpu.VMEM((PS[0],), jnp.float32),
            pltpu.VMEM((PS[1],), jnp.float32),
            pltpu.VMEM((DPS * BATCH,), jnp.float32),
            pltpu.SemaphoreType.DMA,
            pltpu.SemaphoreType.DMA,
        ],
        compiler_params=pltpu.CompilerParams(
            use_tc_tiling_on_sc=True, needs_layout_passes=False
        ),
    )
    k2 = pl.kernel(
        _combine_body,
        out_type=jax.ShapeDtypeStruct((DIM, BATCH), jnp.float32),
        mesh=mesh,
        scratch_types=[
            pltpu.VMEM((DIM, 128), jnp.float32),
            pltpu.VMEM((DIM, 128), jnp.float32),
            pltpu.VMEM((DIM, 128), jnp.float32),
        ],
        compiler_params=pltpu.CompilerParams(
            use_tc_tiling_on_sc=True, needs_layout_passes=False
        ),
    )
    part = k1(idx2d, wt)
    return k2(part)


def kernel(xs, W):
    idx2d = xs[:, :, 0].astype(jnp.int32)
    wt = jnp.transpose(W, (0, 2, 1))
    out_t = _multi_embedding(idx2d, wt)
    return jnp.transpose(out_t)


# speedup vs baseline: 1.4069x; 1.0436x over previous
import jax
import jax.numpy as jnp
from jax import lax
from jax.experimental import pallas as pl
from jax.experimental.pallas import tpu as pltpu, tpu_sc as plsc

NUM_FIELDS = 26
VOCAB = 100000
DIM = 64
BATCH = 4096

NC, NS, L = 2, 16, 16
FPC = NUM_FIELDS // NC
DPS = DIM // NS
NPIECE = 2
PS = (50048, 49952)
PO = (0, 50048)
NPOS = FPC * DPS * NPIECE
CHUNKS = BATCH // L
UNROLL = 8


def _acc_body(idx_hbm, wt_hbm, part_hbm, idxv, b0, b1, acc, s0, s1):
    cid = lax.axis_index("c")
    sid = lax.axis_index("s")
    bufs = (b0, b1)
    sems = (s0, s1)

    def zstep(i, _):
        acc[pl.ds(i * L, L)] = jnp.zeros((L,), jnp.float32)
        return 0

    lax.fori_loop(0, DPS * BATCH // L, zstep, 0)

    iota = lax.iota(jnp.int32, L)

    def src(pos, k):
        fi = pos // (DPS * NPIECE)
        dslot = (pos // NPIECE) % DPS
        f = cid * FPC + fi
        d = sid * DPS + dslot
        return wt_hbm.at[f, d, pl.ds(PO[k], PS[k])]

    pltpu.async_copy(src(0, 0), bufs[0], sems[0])

    def compute(dslot, k):
        lo = jnp.int32(PO[k])
        hi = jnp.int32(PO[k] + PS[k])
        base_f = dslot * BATCH
        buf = bufs[k]

        def kstep(kk, _):
            for j in range(UNROLL):
                b0_ = kk * (L * UNROLL) + j * L
                v = idxv[pl.ds(b0_, L)]
                fidx = iota + (base_f + b0_)
                if k == 0:
                    m = v < hi
                else:
                    m = v >= lo
                col = jnp.where(m, v - lo, 0)
                val = plsc.load_gather(buf, [col], mask=m)
                plsc.addupdate_scatter(acc, [fidx], val, mask=m)
            return 0

        lax.fori_loop(0, CHUNKS // UNROLL, kstep, 0)

    def pos_step(pos, _):
        fi = pos // (DPS * NPIECE)
        piece = pos % NPIECE
        f = cid * FPC + fi

        @pl.when(pos % (DPS * NPIECE) == 0)
        def _():
            pltpu.sync_copy(idx_hbm.at[f], idxv)

        for k in range(NPIECE):
            @pl.when(piece == k)
            def _(k=k):
                dslot = (pos // NPIECE) % DPS

                @pl.when(pos + 1 < NPOS)
                def _():
                    pltpu.async_copy(
                        src(pos + 1, 1 - k), bufs[1 - k], sems[1 - k]
                    )

                pltpu.make_async_copy(src(pos, k), bufs[k], sems[k]).wait()
                compute(dslot, k)

        return 0

    lax.fori_loop(0, NPOS, pos_step, 0)

    for dslot in range(DPS):
        d = sid * DPS + dslot
        pltpu.sync_copy(
            acc.at[pl.ds(dslot * BATCH, BATCH)], part_hbm.at[cid, d]
        )


def _combine_body(part_hbm, out_hbm, p0v, p1v, ov):
    wid = lax.axis_index("s") * NC + lax.axis_index("c")
    scale = jnp.float32(1.0 / NUM_FIELDS)
    pltpu.sync_copy(part_hbm.at[0, :, pl.ds(wid * 128, 128)], p0v)
    pltpu.sync_copy(part_hbm.at[1, :, pl.ds(wid * 128, 128)], p1v)

    def rstep(r, _):
        for j in range(128 // L):
            s = pl.ds(j * L, L)
            ov[r, s] = (p0v[r, s] + p1v[r, s]) * scale
        return 0

    lax.fori_loop(0, DIM, rstep, 0)
    pltpu.sync_copy(ov, out_hbm.at[:, pl.ds(wid * 128, 128)])


@jax.jit
def _multi_embedding(idx2d, wt):
    mesh = plsc.VectorSubcoreMesh(core_axis_name="c", subcore_axis_name="s")
    k1 = pl.kernel(
        _acc_body,
        out_type=jax.ShapeDtypeStruct((NC, DIM, BATCH), jnp.float32),
        mesh=mesh,
        scratch_types=[
            pltpu.VMEM((BATCH,), jnp.int32),
            pltpu.VMEM((PS[0],), jnp.float32),
            pltpu.VMEM((PS[1],), jnp.float32),
            pltpu.VMEM((DPS * BATCH,), jnp.float32),
            pltpu.SemaphoreType.DMA,
            pltpu.SemaphoreType.DMA,
        ],
        compiler_params=pltpu.CompilerParams(
            use_tc_tiling_on_sc=True, needs_layout_passes=False
        ),
    )
    k2 = pl.kernel(
        _combine_body,
        out_type=jax.ShapeDtypeStruct((DIM, BATCH), jnp.float32),
        mesh=mesh,
        scratch_types=[
            pltpu.VMEM((DIM, 128), jnp.float32),
            pltpu.VMEM((DIM, 128), jnp.float32),
            pltpu.VMEM((DIM, 128), jnp.float32),
        ],
        compiler_params=pltpu.CompilerParams(
            use_tc_tiling_on_sc=True, needs_layout_passes=False
        ),
    )
    part = k1(idx2d, wt)
    return k2(part)


def kernel(xs, W):
    idx2d = xs[:, :, 0].astype(jnp.int32)
    wt = jnp.transpose(W, (0, 2, 1))
    out_t = _multi_embedding(idx2d, wt)
    return jnp.transpose(out_t)
